# Initial kernel scaffold; baseline (speedup 1.0000x reference)
import jax
import jax.numpy as jnp
from jax.experimental import pallas as pl


def kernel(x, weight):
    raise NotImplementedError("write your pallas kernel here")


# speedup vs baseline: 5.2714x; 5.2714x over previous
import functools

import jax
import jax.numpy as jnp
from jax import lax
from jax.experimental import pallas as pl
from jax.experimental.pallas import tpu as pltpu
from jax.experimental.pallas import tpu_sc as plsc

_NC = 2
_NS = 16
_NW = _NC * _NS
_L = 16


def _make_sc_gather(n_idx: int, d: int, k: int):
    per_tile = n_idx // _NW
    n_chunks = per_tile // k
    assert per_tile * _NW == n_idx and n_chunks * k == per_tile

    mesh = plsc.VectorSubcoreMesh(
        core_axis_name="c", subcore_axis_name="s", num_cores=_NC,
        num_subcores=_NS)

    @functools.partial(
        pl.kernel,
        out_type=jax.ShapeDtypeStruct((n_idx * d,), jnp.float32),
        mesh=mesh,
        compiler_params=pltpu.CompilerParams(needs_layout_passes=False),
        scratch_types=[
            pltpu.VMEM((k,), jnp.int32),
            pltpu.VMEM((k,), jnp.int32),
            pltpu.VMEM((k * d,), jnp.float32),
            pltpu.VMEM((k * d,), jnp.float32),
            pltpu.VMEM((32,), jnp.float32),
            pltpu.SemaphoreType.DMA,
            pltpu.SemaphoreType.DMA,
        ],
    )
    def sc_gather(x_hbm, w_hbm, out_hbm, idx0_v, idx1_v, out0_v, out1_v,
                  w_v, in_sem, out_sem):
        idx_bufs = [idx0_v, idx1_v]
        out_bufs = [out0_v, out1_v]
        wid = lax.axis_index("s") * _NC + lax.axis_index("c")
        base = wid * per_tile
        pltpu.sync_copy(w_hbm, w_v)
        iota = lax.iota(jnp.int32, _L)
        iota_d = iota * d

        def compute(buf):
            def inner(i, _):
                idx = idx_bufs[buf][pl.ds(i * _L, _L)]
                idx_base = idx * d
                sbase = iota_d + i * (_L * d)
                for dd in range(d):
                    vals = plsc.load_gather(w_v, [idx_base + dd])
                    plsc.store_scatter(out_bufs[buf], [sbase + dd], vals)
                return 0
            lax.fori_loop(0, k // _L, inner, 0, unroll=4)

        in_copies = [
            pltpu.async_copy(x_hbm.at[pl.ds(base, k)], idx_bufs[0], in_sem)]
        out_copies = [None, None]
        for c in range(n_chunks):
            buf = c % 2
            in_copies.pop(0).wait()
            if c + 1 < n_chunks:
                in_copies.append(pltpu.async_copy(
                    x_hbm.at[pl.ds(base + (c + 1) * k, k)],
                    idx_bufs[1 - buf], in_sem))
            if out_copies[buf] is not None:
                out_copies[buf].wait()
            compute(buf)
            out_copies[buf] = pltpu.async_copy(
                out_bufs[buf],
                out_hbm.at[pl.ds((base + c * k) * d, k * d)], out_sem)

        for cp in out_copies:
            if cp is not None:
                cp.wait()

    return sc_gather


_N_IDX = 16384 * 200
_D = 4
_K = 12800


@functools.lru_cache(maxsize=None)
def _sc_gather_fn():
    return _make_sc_gather(_N_IDX, _D, _K)


@jax.jit
def kernel(x, weight):
    xf = x.reshape(-1)
    wf = jnp.pad(weight.reshape(-1), (0, 32 - weight.size))
    out = _sc_gather_fn()(xf, wf)
    return out.reshape(x.shape + (weight.shape[1],))


# speedup vs baseline: 87.7166x; 16.6401x over previous
import functools

import jax
import jax.numpy as jnp
from jax import lax
from jax.experimental import pallas as pl
from jax.experimental.pallas import tpu as pltpu
from jax.experimental.pallas import tpu_sc as plsc

_NC = 2
_NS = 16
_NW = _NC * _NS
_L = 16

_NI = 16384
_NJ = 200
_D = 4
_Q = 4
_KI = _NI // _Q
_KO = _KI * _D
_UNITS_PER_TILE = _NJ * _Q // _NW


def _make_sc_gather():
    mesh = plsc.VectorSubcoreMesh(
        core_axis_name="c", subcore_axis_name="s", num_cores=_NC,
        num_subcores=_NS)

    @functools.partial(
        pl.kernel,
        out_type=jax.ShapeDtypeStruct((_NI * _NJ * _D,), jnp.float32),
        mesh=mesh,
        compiler_params=pltpu.CompilerParams(needs_layout_passes=False),
        scratch_types=[
            pltpu.VMEM((_KI,), jnp.int32),
            pltpu.VMEM((_KI,), jnp.int32),
            pltpu.VMEM((_KO,), jnp.float32),
            pltpu.VMEM((_KO,), jnp.float32),
            pltpu.VMEM((32,), jnp.float32),
            pltpu.SemaphoreType.DMA,
            pltpu.SemaphoreType.DMA,
        ],
    )
    def sc_gather(xt_hbm, wc_hbm, out_hbm, idx0_v, idx1_v, out0_v, out1_v,
                  w_v, in_sem, out_sem):
        idx_bufs = [idx0_v, idx1_v]
        out_bufs = [out0_v, out1_v]
        wid = lax.axis_index("s") * _NC + lax.axis_index("c")
        u0 = wid * _UNITS_PER_TILE
        pltpu.sync_copy(wc_hbm, w_v)

        def in_slice(n):
            u = u0 + n
            j = u // _Q
            q = u % _Q
            return xt_hbm.at[j, pl.ds(q * _KI, _KI)]

        def out_slice(n):
            u = u0 + n
            j = u // _Q
            q = u % _Q
            return out_hbm.at[pl.ds(j * (_KO * _Q) + q * _KO, _KO)]

        def compute(buf):
            def inner(m, _):
                for gg in range(8):
                    idx = idx_bufs[buf][pl.ds(m * 128 + gg * _L, _L)]
                    for dd in range(_D):
                        vals = plsc.load_gather(w_v, [idx + dd * 8])
                        out_bufs[buf][
                            pl.ds(m * 512 + dd * 128 + gg * _L, _L)] = vals
                return 0
            lax.fori_loop(0, _KI // 128, inner, 0)

        in_copies = [pltpu.async_copy(in_slice(0), idx_bufs[0], in_sem)]
        out_copies = [None, None]
        for n in range(_UNITS_PER_TILE):
            buf = n % 2
            in_copies.pop(0).wait()
            if n + 1 < _UNITS_PER_TILE:
                in_copies.append(pltpu.async_copy(
                    in_slice(n + 1), idx_bufs[1 - buf], in_sem))
            if out_copies[buf] is not None:
                out_copies[buf].wait()
            compute(buf)
            out_copies[buf] = pltpu.async_copy(
                out_bufs[buf], out_slice(n), out_sem)

        for cp in out_copies:
            if cp is not None:
                cp.wait()

    return sc_gather


@functools.lru_cache(maxsize=None)
def _sc_gather_fn():
    return _make_sc_gather()


@jax.jit
def kernel(x, weight):
    wc = jnp.pad(weight, ((0, 8 - weight.shape[0]), (0, 0))).T.reshape(-1)
    f = _sc_gather_fn()(x.T, wc)
    return (f.reshape(_NJ, _NI // 128, _D, 128)
            .transpose(1, 3, 0, 2)
            .reshape(_NI, _NJ, _D))
